# Initial kernel scaffold; baseline (speedup 1.0000x reference)
import functools

import jax
import jax.numpy as jnp
from jax import lax
from jax.experimental import pallas as pl
from jax.experimental.pallas import tpu as pltpu, tpu_sc as plsc

N, E, F_IN, H, C = 10000, 320000, 128, 256, 40

NC, NS = 2, 16
CHUNK = 128
N_PAD = 10240
ROWS_PER_TILE = N_PAD // NS
E_PAD = 32 * 79 * CHUNK
AGG_CHUNKS = E_PAD // NS // CHUNK
DEG_CHUNKS = E_PAD // (NC * NS) // CHUNK
HH = H // NC

_sc_mesh = plsc.VectorSubcoreMesh(core_axis_name="c", subcore_axis_name="s")


@functools.partial(
    pl.kernel,
    out_type=jax.ShapeDtypeStruct((NC, N_PAD, 8), jnp.float32),
    mesh=_sc_mesh,
    scratch_types=[
        pltpu.VMEM((DEG_CHUNKS, CHUNK), jnp.int32),
        pltpu.VMEM((CHUNK, 8), jnp.float32),
        pltpu.VMEM((ROWS_PER_TILE, 8), jnp.float32),
        pltpu.VMEM_SHARED((N_PAD, 8), jnp.float32),
    ],
)
def _deg_kernel(dst32_hbm, ones_hbm, zeros_hbm, out_hbm,
                dst_v, ones_v, stage_v, deg_sh):
    c = lax.axis_index("c")
    s = lax.axis_index("s")
    w = c * NS + s
    r0 = s * ROWS_PER_TILE
    pltpu.sync_copy(zeros_hbm.at[pl.ds(r0, ROWS_PER_TILE)], stage_v)
    pltpu.sync_copy(stage_v, deg_sh.at[pl.ds(r0, ROWS_PER_TILE)])
    pltpu.sync_copy(ones_hbm, ones_v)
    pltpu.sync_copy(dst32_hbm.at[w], dst_v)
    plsc.subcore_barrier()

    def body(j, carry):
        pltpu.sync_copy(ones_v, deg_sh.at[dst_v.at[j]], add=True)
        return carry

    lax.fori_loop(0, DEG_CHUNKS, body, 0)
    plsc.subcore_barrier()
    pltpu.sync_copy(deg_sh.at[pl.ds(r0, ROWS_PER_TILE)], stage_v)
    pltpu.sync_copy(stage_v, out_hbm.at[c].at[pl.ds(r0, ROWS_PER_TILE)])


@functools.partial(
    pl.kernel,
    out_type=jax.ShapeDtypeStruct((NC * N_PAD, HH), jnp.float32),
    mesh=_sc_mesh,
    scratch_types=[
        pltpu.VMEM((AGG_CHUNKS, CHUNK), jnp.int32),
        pltpu.VMEM((AGG_CHUNKS, CHUNK), jnp.int32),
        pltpu.VMEM((CHUNK, HH), jnp.float32),
        pltpu.VMEM_SHARED((N_PAD, HH), jnp.float32),
        pltpu.SemaphoreType.DMA,
    ],
)
def _agg_kernel(y_hbm, src_hbm, dst_hbm, out_hbm,
                src_v, dst_v, rows_v, acc_sh, sem):
    c = lax.axis_index("c")
    s = lax.axis_index("s")
    r0 = s * ROWS_PER_TILE
    ybase = c * N_PAD + r0
    for k in range(ROWS_PER_TILE // CHUNK):
        pltpu.sync_copy(y_hbm.at[pl.ds(ybase + k * CHUNK, CHUNK)], rows_v)
        pltpu.sync_copy(rows_v, acc_sh.at[pl.ds(r0 + k * CHUNK, CHUNK)])
    pltpu.sync_copy(src_hbm.at[c * NS + s], src_v)
    pltpu.sync_copy(dst_hbm.at[s], dst_v)
    plsc.subcore_barrier()

    def body(j, carry):
        pltpu.async_copy(y_hbm.at[src_v.at[j]], rows_v, sem).wait()
        pltpu.sync_copy(rows_v, acc_sh.at[dst_v.at[j]], add=True)
        return carry

    lax.fori_loop(0, AGG_CHUNKS, body, 0)
    plsc.subcore_barrier()
    for k in range(ROWS_PER_TILE // CHUNK):
        pltpu.sync_copy(acc_sh.at[pl.ds(r0 + k * CHUNK, CHUNK)], rows_v)
        pltpu.sync_copy(rows_v, out_hbm.at[pl.ds(ybase + k * CHUNK, CHUNK)])


BN = 256


def _pre_body(x_ref, w1_ref, deg_ref, y_ref, dinv_ref):
    dinv = lax.rsqrt(jnp.maximum(deg_ref[...], 1.0))
    xw = jnp.dot(x_ref[...], w1_ref[...], preferred_element_type=jnp.float32)
    y = xw * dinv
    y_ref[0] = y[:, :HH]
    y_ref[1] = y[:, HH:]
    dinv_ref[...] = dinv


def _mid_body(agg_ref, dinv_ref, b1_ref, w2_ref, y_ref):
    dinv = dinv_ref[...]
    a = jnp.concatenate([agg_ref[0], agg_ref[1]], axis=1)
    h = jnp.maximum(a * dinv + b1_ref[...], 0.0)
    y = jnp.dot(h, w2_ref[...], preferred_element_type=jnp.float32) * dinv
    y_ref[0] = y[:, :HH]
    y_ref[1] = y[:, HH:]


def _post_body(agg_ref, dinv_ref, b2_ref, wl_ref, bl_ref, out_ref):
    dinv = dinv_ref[...]
    a = jnp.concatenate([agg_ref[0], agg_ref[1]], axis=1)
    h = jnp.maximum(a * dinv + b2_ref[...], 0.0)
    out_ref[...] = (
        jnp.dot(h, wl_ref[...], preferred_element_type=jnp.float32) + bl_ref[...]
    )


_GRID = (N_PAD // BN,)
_split_spec = pl.BlockSpec((2, BN, HH), lambda i: (0, i, 0))
_col_spec = pl.BlockSpec((BN, 1), lambda i: (i, 0))


def _pre_call(x, w1, deg):
    return pl.pallas_call(
        _pre_body,
        grid=_GRID,
        in_specs=[
            pl.BlockSpec((BN, F_IN), lambda i: (i, 0)),
            pl.BlockSpec((F_IN, H), lambda i: (0, 0)),
            _col_spec,
        ],
        out_specs=[_split_spec, _col_spec],
        out_shape=[
            jax.ShapeDtypeStruct((2, N_PAD, HH), jnp.float32),
            jax.ShapeDtypeStruct((N_PAD, 1), jnp.float32),
        ],
    )(x, w1, deg)


def _mid_call(agg, dinv, b1, w2):
    return pl.pallas_call(
        _mid_body,
        grid=_GRID,
        in_specs=[
            _split_spec,
            _col_spec,
            pl.BlockSpec((1, H), lambda i: (0, 0)),
            pl.BlockSpec((H, H), lambda i: (0, 0)),
        ],
        out_specs=_split_spec,
        out_shape=jax.ShapeDtypeStruct((2, N_PAD, HH), jnp.float32),
    )(agg, dinv, b1, w2)


def _post_call(agg, dinv, b2, wl, bl):
    return pl.pallas_call(
        _post_body,
        grid=_GRID,
        in_specs=[
            _split_spec,
            _col_spec,
            pl.BlockSpec((1, H), lambda i: (0, 0)),
            pl.BlockSpec((H, 128), lambda i: (0, 0)),
            pl.BlockSpec((1, 128), lambda i: (0, 0)),
        ],
        out_specs=pl.BlockSpec((BN, 128), lambda i: (i, 0)),
        out_shape=jax.ShapeDtypeStruct((N_PAD, 128), jnp.float32),
    )(agg, dinv, b2, wl, bl)


def kernel(x, edge_index, W1, b1, W2, b2, Wl, bl):
    src = edge_index[0]
    dst = edge_index[1]
    pad = E_PAD - E
    src_p = jnp.concatenate([src, jnp.zeros((pad,), jnp.int32)])
    dst_p = jnp.concatenate([dst, jnp.full((pad,), N, jnp.int32)])

    dst32 = dst_p.reshape(NC * NS, DEG_CHUNKS, CHUNK)
    dst16 = dst_p.reshape(NS, AGG_CHUNKS, CHUNK)
    src16 = src_p.reshape(NS, AGG_CHUNKS, CHUNK)
    src_b = jnp.concatenate([src16, src16 + N_PAD], axis=0)

    ones8 = jnp.ones((CHUNK, 8), jnp.float32)
    zeros8 = jnp.zeros((N_PAD, 8), jnp.float32)

    deg2 = _deg_kernel(dst32, ones8, zeros8)
    deg = deg2[0, :, :1] + deg2[1, :, :1] + 1.0

    x_pad = jnp.pad(x, ((0, N_PAD - N), (0, 0)))
    wl_pad = jnp.pad(Wl, ((0, 0), (0, 128 - C)))
    bl_pad = jnp.pad(bl, ((0, 128 - C),)).reshape(1, 128)
    b1_2 = b1.reshape(1, H)
    b2_2 = b2.reshape(1, H)

    y1, dinv = _pre_call(x_pad, W1, deg)
    agg1 = _agg_kernel(y1.reshape(NC * N_PAD, HH), src_b, dst16)
    y2 = _mid_call(agg1.reshape(NC, N_PAD, HH), dinv, b1_2, W2)
    agg2 = _agg_kernel(y2.reshape(NC * N_PAD, HH), src_b, dst16)
    out = _post_call(agg2.reshape(NC, N_PAD, HH), dinv, b2_2, wl_pad, bl_pad)
    return out[:N, :C]


# speedup vs baseline: 8.5863x; 8.5863x over previous
import functools

import jax
import jax.numpy as jnp
from jax import lax
from jax.experimental import pallas as pl
from jax.experimental.pallas import tpu as pltpu, tpu_sc as plsc

N, E, F_IN, H, C = 10000, 320000, 128, 256, 40

NC, NS = 2, 16
CHUNK = 128
N_PAD = 10240
ROWS_PER_TILE = N_PAD // NS
E_PAD = 32 * 79 * CHUNK
AGG_CHUNKS = E_PAD // NS // CHUNK
DEG_CHUNKS = E_PAD // (NC * NS) // CHUNK
HH = H // NC

_sc_mesh = plsc.VectorSubcoreMesh(core_axis_name="c", subcore_axis_name="s")


@functools.partial(
    pl.kernel,
    out_type=jax.ShapeDtypeStruct((NC * N_PAD, 128), jnp.float32),
    mesh=_sc_mesh,
    scratch_types=[
        pltpu.VMEM((CHUNK,), jnp.int32),
        pltpu.VMEM((CHUNK, 128), jnp.float32),
        pltpu.VMEM((CHUNK, 128), jnp.float32),
        pltpu.VMEM_SHARED((N_PAD, 128), jnp.float32),
    ],
)
def _deg_kernel(dst_flat_hbm, ones_hbm, zeros_hbm, out_hbm,
                dst_v, ones_v, stage_v, deg_sh):
    c = lax.axis_index("c")
    s = lax.axis_index("s")
    w = c * NS + s
    r0 = s * ROWS_PER_TILE
    pltpu.sync_copy(zeros_hbm, stage_v)
    for k in range(ROWS_PER_TILE // CHUNK):
        pltpu.sync_copy(stage_v, deg_sh.at[pl.ds(r0 + k * CHUNK, CHUNK)])
    pltpu.sync_copy(ones_hbm, ones_v)
    e0 = w * DEG_CHUNKS * CHUNK
    plsc.subcore_barrier()

    @pl.loop(0, DEG_CHUNKS)
    def _deg_loop(j):
        pltpu.sync_copy(dst_flat_hbm.at[pl.ds(e0 + j * CHUNK, CHUNK)], dst_v)
        pltpu.sync_copy(ones_v, deg_sh.at[dst_v], add=True)
    plsc.subcore_barrier()
    for k in range(ROWS_PER_TILE // CHUNK):
        pltpu.sync_copy(deg_sh.at[pl.ds(r0 + k * CHUNK, CHUNK)], stage_v)
        pltpu.sync_copy(
            stage_v, out_hbm.at[pl.ds(c * N_PAD + r0 + k * CHUNK, CHUNK)])


@functools.partial(
    pl.kernel,
    out_type=jax.ShapeDtypeStruct((NC * N_PAD, HH), jnp.float32),
    mesh=_sc_mesh,
    scratch_types=[
        pltpu.VMEM((CHUNK,), jnp.int32),
        pltpu.VMEM((CHUNK,), jnp.int32),
        pltpu.VMEM((CHUNK, HH), jnp.float32),
        pltpu.VMEM_SHARED((N_PAD, HH), jnp.float32),
        pltpu.SemaphoreType.DMA,
    ],
)
def _agg_kernel(y_hbm, src_hbm, dst_hbm, out_hbm,
                src_v, dst_v, rows_v, acc_sh, sem):
    c = lax.axis_index("c")
    s = lax.axis_index("s")
    r0 = s * ROWS_PER_TILE
    ybase = c * N_PAD + r0
    for k in range(ROWS_PER_TILE // CHUNK):
        pltpu.sync_copy(y_hbm.at[pl.ds(ybase + k * CHUNK, CHUNK)], rows_v)
        pltpu.sync_copy(rows_v, acc_sh.at[pl.ds(r0 + k * CHUNK, CHUNK)])
    src_e0 = (c * NS + s) * AGG_CHUNKS * CHUNK
    dst_e0 = s * AGG_CHUNKS * CHUNK
    plsc.subcore_barrier()

    @pl.loop(0, AGG_CHUNKS)
    def _agg_loop(j):
        pltpu.sync_copy(src_hbm.at[pl.ds(src_e0 + j * CHUNK, CHUNK)], src_v)
        pltpu.sync_copy(dst_hbm.at[pl.ds(dst_e0 + j * CHUNK, CHUNK)], dst_v)
        pltpu.async_copy(y_hbm.at[src_v], rows_v, sem).wait()
        pltpu.sync_copy(rows_v, acc_sh.at[dst_v], add=True)
    plsc.subcore_barrier()
    for k in range(ROWS_PER_TILE // CHUNK):
        pltpu.sync_copy(acc_sh.at[pl.ds(r0 + k * CHUNK, CHUNK)], rows_v)
        pltpu.sync_copy(rows_v, out_hbm.at[pl.ds(ybase + k * CHUNK, CHUNK)])


BN = 256


def _pre_body(x_ref, w1_ref, deg_ref, y_ref, dinv_ref):
    dinv = lax.rsqrt(jnp.maximum(deg_ref[...], 1.0))
    xw = jnp.dot(x_ref[...], w1_ref[...], preferred_element_type=jnp.float32)
    y = xw * dinv
    y_ref[0] = y[:, :HH]
    y_ref[1] = y[:, HH:]
    dinv_ref[...] = dinv


def _mid_body(agg_ref, dinv_ref, b1_ref, w2_ref, y_ref):
    dinv = dinv_ref[...]
    a = jnp.concatenate([agg_ref[0], agg_ref[1]], axis=1)
    h = jnp.maximum(a * dinv + b1_ref[...], 0.0)
    y = jnp.dot(h, w2_ref[...], preferred_element_type=jnp.float32) * dinv
    y_ref[0] = y[:, :HH]
    y_ref[1] = y[:, HH:]


def _post_body(agg_ref, dinv_ref, b2_ref, wl_ref, bl_ref, out_ref):
    dinv = dinv_ref[...]
    a = jnp.concatenate([agg_ref[0], agg_ref[1]], axis=1)
    h = jnp.maximum(a * dinv + b2_ref[...], 0.0)
    out_ref[...] = (
        jnp.dot(h, wl_ref[...], preferred_element_type=jnp.float32) + bl_ref[...]
    )


_GRID = (N_PAD // BN,)
_split_spec = pl.BlockSpec((2, BN, HH), lambda i: (0, i, 0))
_col_spec = pl.BlockSpec((BN, 1), lambda i: (i, 0))


def _pre_call(x, w1, deg):
    return pl.pallas_call(
        _pre_body,
        grid=_GRID,
        in_specs=[
            pl.BlockSpec((BN, F_IN), lambda i: (i, 0)),
            pl.BlockSpec((F_IN, H), lambda i: (0, 0)),
            _col_spec,
        ],
        out_specs=[_split_spec, _col_spec],
        out_shape=[
            jax.ShapeDtypeStruct((2, N_PAD, HH), jnp.float32),
            jax.ShapeDtypeStruct((N_PAD, 1), jnp.float32),
        ],
    )(x, w1, deg)


def _mid_call(agg, dinv, b1, w2):
    return pl.pallas_call(
        _mid_body,
        grid=_GRID,
        in_specs=[
            _split_spec,
            _col_spec,
            pl.BlockSpec((1, H), lambda i: (0, 0)),
            pl.BlockSpec((H, H), lambda i: (0, 0)),
        ],
        out_specs=_split_spec,
        out_shape=jax.ShapeDtypeStruct((2, N_PAD, HH), jnp.float32),
    )(agg, dinv, b1, w2)


def _post_call(agg, dinv, b2, wl, bl):
    return pl.pallas_call(
        _post_body,
        grid=_GRID,
        in_specs=[
            _split_spec,
            _col_spec,
            pl.BlockSpec((1, H), lambda i: (0, 0)),
            pl.BlockSpec((H, 128), lambda i: (0, 0)),
            pl.BlockSpec((1, 128), lambda i: (0, 0)),
        ],
        out_specs=pl.BlockSpec((BN, 128), lambda i: (i, 0)),
        out_shape=jax.ShapeDtypeStruct((N_PAD, 128), jnp.float32),
    )(agg, dinv, b2, wl, bl)


def kernel(x, edge_index, W1, b1, W2, b2, Wl, bl):
    src = edge_index[0]
    dst = edge_index[1]
    pad = E_PAD - E
    src_p = jnp.concatenate([src, jnp.zeros((pad,), jnp.int32)])
    dst_p = jnp.concatenate([dst, jnp.full((pad,), N, jnp.int32)])

    src_b = jnp.concatenate([src_p, src_p + N_PAD])

    ones16 = jnp.ones((CHUNK, 128), jnp.float32)
    zeros16 = jnp.zeros((CHUNK, 128), jnp.float32)

    deg2 = _deg_kernel(dst_p, ones16, zeros16)
    deg = deg2[:N_PAD, :1] + deg2[N_PAD:, :1] + 1.0

    x_pad = jnp.pad(x, ((0, N_PAD - N), (0, 0)))
    wl_pad = jnp.pad(Wl, ((0, 0), (0, 128 - C)))
    bl_pad = jnp.pad(bl, ((0, 128 - C),)).reshape(1, 128)
    b1_2 = b1.reshape(1, H)
    b2_2 = b2.reshape(1, H)

    y1, dinv = _pre_call(x_pad, W1, deg)
    agg1 = _agg_kernel(y1.reshape(NC * N_PAD, HH), src_b, dst_p)
    y2 = _mid_call(agg1.reshape(NC, N_PAD, HH), dinv, b1_2, W2)
    agg2 = _agg_kernel(y2.reshape(NC * N_PAD, HH), src_b, dst_p)
    out = _post_call(agg2.reshape(NC, N_PAD, HH), dinv, b2_2, wl_pad, bl_pad)
    return out[:N, :C]
